# Initial kernel scaffold; baseline (speedup 1.0000x reference)
import jax
import jax.numpy as jnp
from jax.experimental import pallas as pl


def kernel(input_ids, attention_mask, fill_vals):
    raise NotImplementedError("write your pallas kernel here")


# speedup vs baseline: 1.3396x; 1.3396x over previous
import jax
import jax.numpy as jnp
from jax.experimental import pallas as pl
from jax.experimental.pallas import tpu as pltpu

_BS = 256


def _oracle_block(ids_ref, halo_ref, mask_ref, fill_ref, out_ref):
    i = pl.program_id(1)
    nb = pl.num_programs(1)
    bs = out_ref.shape[1]
    v = out_ref.shape[2]
    v0 = fill_ref[0]
    v1 = fill_ref[1]

    last = jnp.sum(mask_ref[...]) - 1

    cur = ids_ref[0]
    is_last = i == nb - 1
    edge = jnp.where(is_last, jnp.full((1, 1), 3, jnp.int32), halo_ref[0][0:1, :])
    tok = jnp.concatenate([cur[1:, :], edge], axis=0)
    start = i * bs
    pos = start + jax.lax.broadcasted_iota(jnp.int32, (bs, 1), 0)
    tok = jnp.where(pos == last, 3, tok)
    hit = pos <= last

    vocab = jax.lax.broadcasted_iota(jnp.int32, (bs, v), 1)
    sel = (vocab == tok) & hit
    out_ref[0] = jnp.where(sel, v1, v0)


def kernel(input_ids, attention_mask, fill_vals):
    b, s = input_ids.shape
    v = 1000
    mask_i32 = attention_mask.astype(jnp.int32).reshape(b, 1, s)
    ids_3d = input_ids.reshape(b, s, 1)
    nb = s // _BS
    grid = (b, nb)
    return pl.pallas_call(
        _oracle_block,
        grid=grid,
        in_specs=[
            pl.BlockSpec((1, _BS, 1), lambda bi, si: (bi, si, 0)),
            pl.BlockSpec((1, _BS, 1), lambda bi, si: (bi, jnp.minimum(si + 1, nb - 1), 0)),
            pl.BlockSpec((1, 1, s), lambda bi, si: (bi, 0, 0)),
            pl.BlockSpec(memory_space=pltpu.SMEM),
        ],
        out_specs=pl.BlockSpec((1, _BS, v), lambda bi, si: (bi, si, 0)),
        out_shape=jax.ShapeDtypeStruct((b, s, v), jnp.float32),
    )(ids_3d, ids_3d, mask_i32, fill_vals)


# speedup vs baseline: 1.5967x; 1.1920x over previous
import jax
import jax.numpy as jnp
from jax.experimental import pallas as pl
from jax.experimental.pallas import tpu as pltpu

_BS = 512


def _oracle_block(ids_ref, halo_ref, mask_ref, fill_ref, out_ref):
    i = pl.program_id(1)
    nb = pl.num_programs(1)
    bs = out_ref.shape[1]
    v = out_ref.shape[2]
    v0 = fill_ref[0]
    v1 = fill_ref[1]

    last = jnp.sum(mask_ref[...]) - 1

    cur = ids_ref[0]
    is_last = i == nb - 1
    edge = jnp.where(is_last, jnp.full((1, 1), 3, jnp.int32), halo_ref[0][0:1, :])
    tok = jnp.concatenate([cur[1:, :], edge], axis=0)
    start = i * bs
    pos = start + jax.lax.broadcasted_iota(jnp.int32, (bs, 1), 0)
    tok = jnp.where(pos == last, 3, tok)
    tok = jnp.where(pos <= last, tok, -1)

    vocab = jax.lax.broadcasted_iota(jnp.int32, (bs, v), 1)
    out_ref[0] = jnp.where(vocab == tok, v1, v0)


def kernel(input_ids, attention_mask, fill_vals):
    b, s = input_ids.shape
    v = 1000
    mask_i32 = attention_mask.astype(jnp.int32).reshape(b, 1, s)
    ids_3d = input_ids.reshape(b, s, 1)
    nb = s // _BS
    grid = (b, nb)
    return pl.pallas_call(
        _oracle_block,
        grid=grid,
        in_specs=[
            pl.BlockSpec((1, _BS, 1), lambda bi, si: (bi, si, 0)),
            pl.BlockSpec((1, _BS, 1), lambda bi, si: (bi, jnp.minimum(si + 1, nb - 1), 0)),
            pl.BlockSpec((1, 1, s), lambda bi, si: (bi, 0, 0)),
            pl.BlockSpec(memory_space=pltpu.SMEM),
        ],
        out_specs=pl.BlockSpec((1, _BS, v), lambda bi, si: (bi, si, 0)),
        out_shape=jax.ShapeDtypeStruct((b, s, v), jnp.float32),
    )(ids_3d, ids_3d, mask_i32, fill_vals)
